# baseline (speedup 1.0000x reference)
import jax
import jax.numpy as jnp
from jax.experimental import pallas as pl
from jax.experimental.pallas import tpu as pltpu

_BR = 512


def _body(x_ref, adj_ref, w_ref, b_ref, keep_ref, out_ref,
          dinv_ref, y_ref, z_ref):
    p = pl.program_id(0)
    t = pl.program_id(1)
    nt = pl.num_programs(1)

    @pl.when(jnp.logical_and(p == 0, t == 0))
    def _init_deg():
        dinv_ref[...] = jnp.ones_like(dinv_ref)

    @pl.when(p == 0)
    def _deg_phase():
        ones_col = jnp.ones((_BR, 1), dtype=jnp.float32)
        dinv_ref[...] += jax.lax.dot_general(
            adj_ref[...], ones_col, (((0,), (0,)), ((), ())),
            preferred_element_type=jnp.float32,
        )

    @pl.when(jnp.logical_and(p == 0, t == nt - 1))
    def _finish_deg():
        dinv = jax.lax.rsqrt(dinv_ref[...])
        dinv_ref[...] = dinv
        y_ref[...] = x_ref[...] * keep_ref[...] * 2.0 * dinv

    @pl.when(p == 1)
    def _mm_phase():
        y_tile = y_ref[pl.ds(t * _BR, _BR), :]
        acc = jax.lax.dot_general(
            adj_ref[...], y_tile, (((0,), (0,)), ((), ())),
            preferred_element_type=jnp.float32,
        )

        @pl.when(t == 0)
        def _first():
            z_ref[...] = y_ref[...] + acc

        @pl.when(t > 0)
        def _rest():
            z_ref[...] += acc

    @pl.when(jnp.logical_and(p == 1, t == nt - 1))
    def _epilogue():
        z = z_ref[...] * dinv_ref[...]
        h = jax.lax.dot_general(
            z, w_ref[...], (((1,), (1,)), ((), ())),
            preferred_element_type=jnp.float32,
            precision=jax.lax.Precision.HIGHEST,
        ) + b_ref[...]
        h = h - jnp.mean(h, axis=0, keepdims=True)
        rnorm = jnp.sqrt(1e-6 + jnp.sum(h * h, axis=1, keepdims=True))
        out_ref[...] = jnp.maximum(h / rnorm, 0.0)


def kernel(x, adj, W, b):
    n, f = x.shape
    keep = jax.random.bernoulli(
        jax.random.key(42), 0.5, x.shape).astype(jnp.float32)
    out = pl.pallas_call(
        _body,
        grid=(2, n // _BR),
        in_specs=[
            pl.BlockSpec((n, f), lambda p, t: (0, 0)),
            pl.BlockSpec((_BR, n), lambda p, t: (t, 0)),
            pl.BlockSpec((f, f), lambda p, t: (0, 0)),
            pl.BlockSpec((1, f), lambda p, t: (0, 0)),
            pl.BlockSpec((n, f), lambda p, t: (0, 0)),
        ],
        out_specs=pl.BlockSpec((n, f), lambda p, t: (0, 0)),
        out_shape=jax.ShapeDtypeStruct((n, f), jnp.float32),
        scratch_shapes=[
            pltpu.VMEM((n, 1), jnp.float32),
            pltpu.VMEM((n, f), jnp.float32),
            pltpu.VMEM((n, f), jnp.float32),
        ],
    )(x, adj, W, b.reshape(1, f), keep)
    return (out, adj)


# speedup vs baseline: 1.0806x; 1.0806x over previous
import jax
import jax.numpy as jnp
from jax.experimental import pallas as pl
from jax.experimental.pallas import tpu as pltpu

_BR = 512


def _body(x_ref, adj_ref, w_ref, b_ref, keep_ref, out_ref,
          dinv_ref, adj_bf):
    t = pl.program_id(0)
    nt = pl.num_programs(0) - 1

    @pl.when(t == 0)
    def _init_deg():
        dinv_ref[...] = jnp.ones_like(dinv_ref)

    @pl.when(t < nt)
    def _deg_and_park():
        adj = adj_ref[...]
        adj_bf[pl.ds(t * _BR, _BR), :] = adj.astype(jnp.bfloat16)
        ones_col = jnp.ones((_BR, 1), dtype=jnp.float32)
        dinv_ref[...] += jax.lax.dot_general(
            adj, ones_col, (((0,), (0,)), ((), ())),
            preferred_element_type=jnp.float32,
        )

    @pl.when(t == nt)
    def _compute():
        dinv = jax.lax.rsqrt(dinv_ref[...])
        y = x_ref[...] * keep_ref[...] * 2.0 * dinv
        z = jax.lax.dot_general(
            adj_bf[...], y.astype(jnp.bfloat16),
            (((0,), (0,)), ((), ())),
            preferred_element_type=jnp.float32,
        ) + y
        z = z * dinv
        h = jax.lax.dot_general(
            z, w_ref[...], (((1,), (1,)), ((), ())),
            preferred_element_type=jnp.float32,
            precision=jax.lax.Precision.HIGHEST,
        ) + b_ref[...]
        h = h - jnp.mean(h, axis=0, keepdims=True)
        rnorm = jnp.sqrt(1e-6 + jnp.sum(h * h, axis=1, keepdims=True))
        out_ref[...] = jnp.maximum(h / rnorm, 0.0)


def kernel(x, adj, W, b):
    n, f = x.shape
    nt = n // _BR
    keep = jax.random.bernoulli(
        jax.random.key(42), 0.5, x.shape).astype(jnp.float32)
    out = pl.pallas_call(
        _body,
        grid=(nt + 1,),
        in_specs=[
            pl.BlockSpec((n, f), lambda t: (0, 0)),
            pl.BlockSpec((_BR, n),
                         lambda t: (jnp.minimum(t, nt - 1), 0)),
            pl.BlockSpec((f, f), lambda t: (0, 0)),
            pl.BlockSpec((1, f), lambda t: (0, 0)),
            pl.BlockSpec((n, f), lambda t: (0, 0)),
        ],
        out_specs=pl.BlockSpec((n, f), lambda t: (0, 0)),
        out_shape=jax.ShapeDtypeStruct((n, f), jnp.float32),
        scratch_shapes=[
            pltpu.VMEM((n, 1), jnp.float32),
            pltpu.VMEM((n, n), jnp.bfloat16),
        ],
    )(x, adj, W, b.reshape(1, f), keep)
    return (out, adj)


# speedup vs baseline: 1.0899x; 1.0087x over previous
import jax
import jax.numpy as jnp
from jax.experimental import pallas as pl
from jax.experimental.pallas import tpu as pltpu

_BR = 1024


def _body(x_ref, adj_ref, w_ref, b_ref, keep_ref, out_ref,
          dinv_ref, adj_bf):
    t = pl.program_id(0)
    nt = pl.num_programs(0) - 1

    @pl.when(t == 0)
    def _init_deg():
        dinv_ref[...] = jnp.ones_like(dinv_ref)

    @pl.when(t < nt)
    def _deg_and_park():
        adj = adj_ref[...]
        adj_bf[pl.ds(t * _BR, _BR), :] = adj.astype(jnp.bfloat16)
        ones_col = jnp.ones((_BR, 1), dtype=jnp.float32)
        dinv_ref[...] += jax.lax.dot_general(
            adj, ones_col, (((0,), (0,)), ((), ())),
            preferred_element_type=jnp.float32,
        )

    @pl.when(t == nt)
    def _compute():
        dinv = jax.lax.rsqrt(dinv_ref[...])
        y = x_ref[...] * keep_ref[...] * 2.0 * dinv
        z = jax.lax.dot_general(
            adj_bf[...], y.astype(jnp.bfloat16),
            (((0,), (0,)), ((), ())),
            preferred_element_type=jnp.float32,
        ) + y
        z = z * dinv
        h = jax.lax.dot_general(
            z, w_ref[...], (((1,), (1,)), ((), ())),
            preferred_element_type=jnp.float32,
            precision=jax.lax.Precision.HIGHEST,
        ) + b_ref[...]
        h = h - jnp.mean(h, axis=0, keepdims=True)
        rnorm = jnp.sqrt(1e-6 + jnp.sum(h * h, axis=1, keepdims=True))
        out_ref[...] = jnp.maximum(h / rnorm, 0.0)


def kernel(x, adj, W, b):
    n, f = x.shape
    nt = n // _BR
    keep = jax.random.bernoulli(
        jax.random.key(42), 0.5, x.shape).astype(jnp.float32)
    out = pl.pallas_call(
        _body,
        grid=(nt + 1,),
        in_specs=[
            pl.BlockSpec((n, f), lambda t: (0, 0)),
            pl.BlockSpec((_BR, n),
                         lambda t: (jnp.minimum(t, nt - 1), 0)),
            pl.BlockSpec((f, f), lambda t: (0, 0)),
            pl.BlockSpec((1, f), lambda t: (0, 0)),
            pl.BlockSpec((n, f), lambda t: (0, 0)),
        ],
        out_specs=pl.BlockSpec((n, f), lambda t: (0, 0)),
        out_shape=jax.ShapeDtypeStruct((n, f), jnp.float32),
        scratch_shapes=[
            pltpu.VMEM((n, 1), jnp.float32),
            pltpu.VMEM((n, n), jnp.bfloat16),
        ],
    )(x, adj, W, b.reshape(1, f), keep)
    return (out, adj)


# speedup vs baseline: 1.5378x; 1.4109x over previous
import jax
import jax.numpy as jnp
from jax.experimental import pallas as pl


def _body(xt_ref, adj_ref, w_ref, b_ref, keept_ref, out_ref):
    adj = adj_ref[...]
    n = adj.shape[0]
    ones_row = jnp.ones((1, n), dtype=jnp.float32)
    deg = jax.lax.dot_general(
        ones_row, adj, (((1,), (0,)), ((), ())),
        preferred_element_type=jnp.float32,
    ) + 1.0
    dinv = jax.lax.rsqrt(deg)

    ytd = xt_ref[...] * keept_ref[...] * 2.0 * dinv
    zt = jax.lax.dot_general(
        ytd, adj, (((1,), (0,)), ((), ())),
        preferred_element_type=jnp.float32,
    )
    zt = (zt + ytd) * dinv

    ht = jax.lax.dot_general(
        w_ref[...], zt, (((1,), (0,)), ((), ())),
        preferred_element_type=jnp.float32,
        precision=jax.lax.Precision.HIGHEST,
    ) + b_ref[...]
    ht = ht - jnp.mean(ht, axis=1, keepdims=True)
    rnorm = jnp.sqrt(1e-6 + jnp.sum(ht * ht, axis=0, keepdims=True))
    out_ref[...] = jnp.maximum(ht / rnorm, 0.0)


def kernel(x, adj, W, b):
    n, f = x.shape
    keep = jax.random.bernoulli(
        jax.random.key(42), 0.5, x.shape).astype(jnp.float32)
    out_t = pl.pallas_call(
        _body,
        out_shape=jax.ShapeDtypeStruct((f, n), jnp.float32),
    )(x.T, adj, W, b.reshape(f, 1), keep.T)
    return (out_t.T, adj)
